# baseline (speedup 1.0000x reference)
import jax
import jax.numpy as jnp
from jax.experimental import pallas as pl
from jax.experimental.pallas import tpu as pltpu

_INDEX = 10
_NCHUNK = 4
_CW = 4096 // _NCHUNK


def _slice_body(x0_hbm, x1_hbm, o0_hbm, o1_hbm, b0, b1, sin, sout):
    ins = []
    for t, (xh, bh) in enumerate(((x0_hbm, b0), (x1_hbm, b1))):
        for k in range(_NCHUNK):
            cp = pltpu.make_async_copy(
                xh.at[_INDEX, :, pl.ds(k * _CW, _CW)],
                bh.at[:, pl.ds(k * _CW, _CW)],
                sin.at[t * _NCHUNK + k],
            )
            cp.start(priority=0)
            ins.append(cp)
    outs = []
    for t, (bh, oh) in enumerate(((b0, o0_hbm), (b1, o1_hbm))):
        for k in range(_NCHUNK):
            ins[t * _NCHUNK + k].wait()
            cp = pltpu.make_async_copy(
                bh.at[:, pl.ds(k * _CW, _CW)],
                oh.at[:, pl.ds(k * _CW, _CW)],
                sout.at[t * _NCHUNK + k],
            )
            cp.start(priority=1)
            outs.append(cp)
    for cp in outs:
        cp.wait()


def kernel(x0, x1):
    B, S, D = x0.shape
    x0t = jnp.transpose(x0, (1, 2, 0))
    x1t = jnp.transpose(x1, (1, 2, 0))
    hbm = pl.BlockSpec(memory_space=pltpu.MemorySpace.HBM)
    o0t, o1t = pl.pallas_call(
        _slice_body,
        in_specs=[hbm, hbm],
        out_specs=[hbm, hbm],
        out_shape=[
            jax.ShapeDtypeStruct((D, B), x0.dtype),
            jax.ShapeDtypeStruct((D, B), x1.dtype),
        ],
        scratch_shapes=[
            pltpu.VMEM((D, B), x0.dtype),
            pltpu.VMEM((D, B), x1.dtype),
            pltpu.SemaphoreType.DMA((2 * _NCHUNK,)),
            pltpu.SemaphoreType.DMA((2 * _NCHUNK,)),
        ],
    )(x0t, x1t)
    return jnp.transpose(o0t, (1, 0)), jnp.transpose(o1t, (1, 0))


# speedup vs baseline: 1.0347x; 1.0347x over previous
import jax
import jax.numpy as jnp
from jax.experimental import pallas as pl
from jax.experimental.pallas import tpu as pltpu

_INDEX = 10
_NCHUNK = 2
_CW = 4096 // _NCHUNK


def _slice_body(x0_hbm, x1_hbm, o0_hbm, o1_hbm, b0, b1, sin, sout):
    ins = []
    for t, (xh, bh) in enumerate(((x0_hbm, b0), (x1_hbm, b1))):
        for k in range(_NCHUNK):
            cp = pltpu.make_async_copy(
                xh.at[_INDEX, :, pl.ds(k * _CW, _CW)],
                bh.at[:, pl.ds(k * _CW, _CW)],
                sin.at[t * _NCHUNK + k],
            )
            cp.start()
            ins.append(cp)
    outs = []
    for t, (bh, oh) in enumerate(((b0, o0_hbm), (b1, o1_hbm))):
        for k in range(_NCHUNK):
            ins[t * _NCHUNK + k].wait()
            cp = pltpu.make_async_copy(
                bh.at[:, pl.ds(k * _CW, _CW)],
                oh.at[:, pl.ds(k * _CW, _CW)],
                sout.at[t * _NCHUNK + k],
            )
            cp.start()
            outs.append(cp)
    for cp in outs:
        cp.wait()


def kernel(x0, x1):
    B, S, D = x0.shape
    x0t = jnp.transpose(x0, (1, 2, 0))
    x1t = jnp.transpose(x1, (1, 2, 0))
    hbm = pl.BlockSpec(memory_space=pltpu.MemorySpace.HBM)
    o0t, o1t = pl.pallas_call(
        _slice_body,
        in_specs=[hbm, hbm],
        out_specs=[hbm, hbm],
        out_shape=[
            jax.ShapeDtypeStruct((D, B), x0.dtype),
            jax.ShapeDtypeStruct((D, B), x1.dtype),
        ],
        scratch_shapes=[
            pltpu.VMEM((D, B), x0.dtype),
            pltpu.VMEM((D, B), x1.dtype),
            pltpu.SemaphoreType.DMA((2 * _NCHUNK,)),
            pltpu.SemaphoreType.DMA((2 * _NCHUNK,)),
        ],
    )(x0t, x1t)
    return jnp.transpose(o0t, (1, 0)), jnp.transpose(o1t, (1, 0))
